# Initial kernel scaffold; baseline (speedup 1.0000x reference)
import jax
import jax.numpy as jnp
from jax.experimental import pallas as pl


def kernel(x, adj_0, adj_1, adj_2, adj_3, mask, emb_table, W, Ws, bias):
    raise NotImplementedError("write your pallas kernel here")


# speedup vs baseline: 4.4790x; 4.4790x over previous
import functools

import jax
import jax.numpy as jnp
from jax import lax
from jax.experimental import pallas as pl

MB = 16

_SELU_SCALE = 1.0507009873554805
_SELU_ALPHA = 1.6732632423543772


def _selu(x):
    return _SELU_SCALE * jnp.where(x > 0, x, _SELU_ALPHA * (jnp.exp(x) - 1.0))


def _body(x_ref, a0_ref, a1_ref, a2_ref, a3_ref, madd_ref, mmul_ref, emb_ref,
          wcat_ref, bias_ref, glo_ref, loc_ref):
    mb = x_ref.shape[0]
    M = mb * 64
    xv = x_ref[...]
    iota = lax.broadcasted_iota(jnp.int32, (mb, 64, 128), 2)
    oh = (xv[:, :, None] == iota).astype(jnp.float32)
    h0 = jnp.dot(oh.reshape(M, 128), emb_ref[...],
                 preferred_element_type=jnp.float32)

    acc = jnp.zeros((M, 384), jnp.float32)
    for b, a_ref in enumerate((a0_ref, a1_ref, a2_ref, a3_ref)):
        A = a_ref[...]
        h = h0
        outs = []
        for l in range(3):
            h3 = h.reshape(mb, 64, 128)
            agg = lax.dot_general(
                A, h3,
                dimension_numbers=(((2,), (1,)), ((0,), (0,))),
                preferred_element_type=jnp.float32)
            hcat = jnp.concatenate([agg.reshape(M, 128), h], axis=1)
            h = (jnp.dot(hcat, wcat_ref[b, l],
                         preferred_element_type=jnp.float32)
                 + bias_ref[b * 3 + l][None, :])
            outs.append(h)
        acc = acc + _selu(jnp.concatenate(outs, axis=1))

    macc = acc * mmul_ref[...]
    loc_ref[...] = (macc + madd_ref[...]).reshape(mb, 64, 384)
    glo_ref[...] = macc.reshape(mb, 64, 384).sum(axis=1)


@functools.partial(jax.jit, static_argnames=("interpret",))
def _run(x, adj_0, adj_1, adj_2, adj_3, madd, mmul, emb_pad, wcat, bias2,
         interpret=False):
    B = x.shape[0]
    grid = (B // MB,)
    blk = lambda *shape: pl.BlockSpec(shape, lambda i: (i,) + (0,) * (len(shape) - 1))
    full = lambda *shape: pl.BlockSpec(shape, lambda i: (0,) * len(shape))
    glo, loc = pl.pallas_call(
        _body,
        grid=grid,
        in_specs=[
            blk(MB, 64),
            blk(MB, 64, 64),
            blk(MB, 64, 64),
            blk(MB, 64, 64),
            blk(MB, 64, 64),
            blk(MB * 64, 1),
            blk(MB * 64, 1),
            full(128, 128),
            full(4, 3, 256, 128),
            full(12, 128),
        ],
        out_specs=[
            blk(MB, 384),
            blk(MB, 64, 384),
        ],
        out_shape=[
            jax.ShapeDtypeStruct((B, 384), jnp.float32),
            jax.ShapeDtypeStruct((B, 64, 384), jnp.float32),
        ],
        interpret=interpret,
    )(x, adj_0, adj_1, adj_2, adj_3, madd, mmul, emb_pad, wcat, bias2)
    return glo, loc


def kernel(x, adj_0, adj_1, adj_2, adj_3, mask, emb_table, W, Ws, bias):
    B, N = x.shape
    emb_pad = jnp.zeros((128, 128), jnp.float32).at[:emb_table.shape[0]].set(
        emb_table)
    wcat = jnp.concatenate([W, Ws], axis=2)
    bias2 = bias.reshape(12, 128)
    mcol = mask.reshape(B * N, 1)
    madd = jnp.where(mcol, 0.0, jnp.nan).astype(jnp.float32)
    mmul = mcol.astype(jnp.float32)
    glo, loc = _run(x.astype(jnp.int32), adj_0, adj_1, adj_2, adj_3,
                    madd, mmul, emb_pad, wcat, bias2)
    return glo, loc.reshape(B * N, 384)


# speedup vs baseline: 4.4796x; 1.0001x over previous
import functools

import jax
import jax.numpy as jnp
from jax import lax
from jax.experimental import pallas as pl

MB = 16

_SELU_SCALE = 1.0507009873554805
_SELU_ALPHA = 1.6732632423543772


def _selu(x):
    return _SELU_SCALE * jnp.where(x > 0, x, _SELU_ALPHA * (jnp.exp(x) - 1.0))


def _body(x_ref, a0_ref, a1_ref, a2_ref, a3_ref, madd_ref, mmul_ref, emb_ref,
          wcat_ref, bias_ref, glo_ref, loc_ref):
    mb = x_ref.shape[0]
    M = mb * 64
    xv = x_ref[...]
    iota = lax.broadcasted_iota(jnp.int32, (mb, 64, 128), 2)
    oh = (xv[:, :, None] == iota).astype(jnp.float32)
    h0 = jnp.dot(oh.reshape(M, 128), emb_ref[...],
                 preferred_element_type=jnp.float32,
                 precision=lax.Precision.DEFAULT)

    acc = jnp.zeros((M, 384), jnp.float32)
    for b, a_ref in enumerate((a0_ref, a1_ref, a2_ref, a3_ref)):
        A = a_ref[...]
        h = h0
        outs = []
        for l in range(3):
            h3 = h.reshape(mb, 64, 128)
            agg = lax.dot_general(
                A, h3,
                dimension_numbers=(((2,), (1,)), ((0,), (0,))),
                preferred_element_type=jnp.float32,
                precision=lax.Precision.DEFAULT)
            hcat = jnp.concatenate([agg.reshape(M, 128), h], axis=1)
            h = (jnp.dot(hcat, wcat_ref[b, l],
                         preferred_element_type=jnp.float32,
                         precision=lax.Precision.DEFAULT)
                 + bias_ref[b * 3 + l][None, :])
            outs.append(h)
        acc = acc + _selu(jnp.concatenate(outs, axis=1))

    macc = acc * mmul_ref[...]
    loc_ref[...] = (macc + madd_ref[...]).reshape(mb, 64, 384)
    glo_ref[...] = macc.reshape(mb, 64, 384).sum(axis=1)


@functools.partial(jax.jit, static_argnames=("interpret",))
def _run(x, adj_0, adj_1, adj_2, adj_3, madd, mmul, emb_pad, wcat, bias2,
         interpret=False):
    B = x.shape[0]
    grid = (B // MB,)
    blk = lambda *shape: pl.BlockSpec(shape, lambda i: (i,) + (0,) * (len(shape) - 1))
    full = lambda *shape: pl.BlockSpec(shape, lambda i: (0,) * len(shape))
    glo, loc = pl.pallas_call(
        _body,
        grid=grid,
        in_specs=[
            blk(MB, 64),
            blk(MB, 64, 64),
            blk(MB, 64, 64),
            blk(MB, 64, 64),
            blk(MB, 64, 64),
            blk(MB * 64, 1),
            blk(MB * 64, 1),
            full(128, 128),
            full(4, 3, 256, 128),
            full(12, 128),
        ],
        out_specs=[
            blk(MB, 384),
            blk(MB, 64, 384),
        ],
        out_shape=[
            jax.ShapeDtypeStruct((B, 384), jnp.float32),
            jax.ShapeDtypeStruct((B, 64, 384), jnp.float32),
        ],
        interpret=interpret,
    )(x, adj_0, adj_1, adj_2, adj_3, madd, mmul, emb_pad, wcat, bias2)
    return glo, loc


def kernel(x, adj_0, adj_1, adj_2, adj_3, mask, emb_table, W, Ws, bias):
    B, N = x.shape
    emb_pad = jnp.zeros((128, 128), jnp.float32).at[:emb_table.shape[0]].set(
        emb_table)
    wcat = jnp.concatenate([W, Ws], axis=2)
    bias2 = bias.reshape(12, 128)
    mcol = mask.reshape(B * N, 1)
    madd = jnp.where(mcol, 0.0, jnp.nan).astype(jnp.float32)
    mmul = mcol.astype(jnp.float32)
    glo, loc = _run(x.astype(jnp.int32), adj_0, adj_1, adj_2, adj_3,
                    madd, mmul, emb_pad, wcat, bias2)
    return glo, loc.reshape(B * N, 384)


# speedup vs baseline: 4.4902x; 1.0024x over previous
import functools

import jax
import jax.numpy as jnp
from jax import lax
from jax.experimental import pallas as pl

MB = 16

_SELU_SCALE = 1.0507009873554805
_SELU_ALPHA = 1.6732632423543772


def _selu(x):
    return _SELU_SCALE * jnp.where(x > 0, x, _SELU_ALPHA * (jnp.exp(x) - 1.0))


def _body(x_ref, a0_ref, a1_ref, a2_ref, a3_ref, madd_ref, mmul_ref, emb_ref,
          wcat_ref, bias_ref, glo_ref, loc_ref):
    mb = x_ref.shape[0]
    M = mb * 64
    xv = x_ref[...]
    iota = lax.broadcasted_iota(jnp.int32, (mb, 64, 128), 2)
    oh = (xv[:, :, None] == iota).astype(jnp.bfloat16)
    h0 = jnp.dot(oh.reshape(M, 128), emb_ref[...].astype(jnp.bfloat16),
                 preferred_element_type=jnp.float32)

    acc = jnp.zeros((M, 384), jnp.float32)
    for b, a_ref in enumerate((a0_ref, a1_ref, a2_ref, a3_ref)):
        A = a_ref[...].astype(jnp.bfloat16)
        h = h0
        outs = []
        for l in range(3):
            hb = h.astype(jnp.bfloat16)
            h3 = hb.reshape(mb, 64, 128)
            agg = lax.dot_general(
                A, h3,
                dimension_numbers=(((2,), (1,)), ((0,), (0,))),
                preferred_element_type=jnp.float32)
            hcat = jnp.concatenate(
                [agg.astype(jnp.bfloat16).reshape(M, 128), hb], axis=1)
            h = (jnp.dot(hcat, wcat_ref[b, l].astype(jnp.bfloat16),
                         preferred_element_type=jnp.float32)
                 + bias_ref[b * 3 + l][None, :])
            outs.append(h)
        acc = acc + _selu(jnp.concatenate(outs, axis=1))

    macc = acc * mmul_ref[...]
    loc_ref[...] = (macc + madd_ref[...]).reshape(mb, 64, 384)
    glo_ref[...] = macc.reshape(mb, 64, 384).sum(axis=1)


@functools.partial(jax.jit, static_argnames=("interpret",))
def _run(x, adj_0, adj_1, adj_2, adj_3, madd, mmul, emb_pad, wcat, bias2,
         interpret=False):
    B = x.shape[0]
    grid = (B // MB,)
    blk = lambda *shape: pl.BlockSpec(shape, lambda i: (i,) + (0,) * (len(shape) - 1))
    full = lambda *shape: pl.BlockSpec(shape, lambda i: (0,) * len(shape))
    glo, loc = pl.pallas_call(
        _body,
        grid=grid,
        in_specs=[
            blk(MB, 64),
            blk(MB, 64, 64),
            blk(MB, 64, 64),
            blk(MB, 64, 64),
            blk(MB, 64, 64),
            blk(MB * 64, 1),
            blk(MB * 64, 1),
            full(128, 128),
            full(4, 3, 256, 128),
            full(12, 128),
        ],
        out_specs=[
            blk(MB, 384),
            blk(MB, 64, 384),
        ],
        out_shape=[
            jax.ShapeDtypeStruct((B, 384), jnp.float32),
            jax.ShapeDtypeStruct((B, 64, 384), jnp.float32),
        ],
        interpret=interpret,
    )(x, adj_0, adj_1, adj_2, adj_3, madd, mmul, emb_pad, wcat, bias2)
    return glo, loc


def kernel(x, adj_0, adj_1, adj_2, adj_3, mask, emb_table, W, Ws, bias):
    B, N = x.shape
    emb_pad = jnp.zeros((128, 128), jnp.float32).at[:emb_table.shape[0]].set(
        emb_table)
    wcat = jnp.concatenate([W, Ws], axis=2)
    bias2 = bias.reshape(12, 128)
    mcol = mask.reshape(B * N, 1)
    madd = jnp.where(mcol, 0.0, jnp.nan).astype(jnp.float32)
    mmul = mcol.astype(jnp.float32)
    glo, loc = _run(x.astype(jnp.int32), adj_0, adj_1, adj_2, adj_3,
                    madd, mmul, emb_pad, wcat, bias2)
    return glo, loc.reshape(B * N, 384)


# speedup vs baseline: 4.8879x; 1.0886x over previous
import functools

import jax
import jax.numpy as jnp
from jax import lax
from jax.experimental import pallas as pl

MB = 32

_SELU_SCALE = 1.0507009873554805
_SELU_ALPHA = 1.6732632423543772


def _selu(x):
    return _SELU_SCALE * jnp.where(x > 0, x, _SELU_ALPHA * (jnp.exp(x) - 1.0))


def _body(x_ref, a0_ref, a1_ref, a2_ref, a3_ref, madd_ref, mmul_ref, emb_ref,
          wcat_ref, bias_ref, glo_ref, loc_ref):
    mb = x_ref.shape[0]
    M = mb * 64
    xv = x_ref[...]
    iota = lax.broadcasted_iota(jnp.int32, (mb, 64, 128), 2)
    oh = (xv[:, :, None] == iota).astype(jnp.bfloat16)
    h0 = jnp.dot(oh.reshape(M, 128), emb_ref[...].astype(jnp.bfloat16),
                 preferred_element_type=jnp.float32)

    acc = jnp.zeros((M, 384), jnp.float32)
    for b, a_ref in enumerate((a0_ref, a1_ref, a2_ref, a3_ref)):
        A = a_ref[...].astype(jnp.bfloat16)
        h = h0
        outs = []
        for l in range(3):
            hb = h.astype(jnp.bfloat16)
            h3 = hb.reshape(mb, 64, 128)
            agg = lax.dot_general(
                A, h3,
                dimension_numbers=(((2,), (1,)), ((0,), (0,))),
                preferred_element_type=jnp.float32)
            hcat = jnp.concatenate(
                [agg.astype(jnp.bfloat16).reshape(M, 128), hb], axis=1)
            h = (jnp.dot(hcat, wcat_ref[b, l].astype(jnp.bfloat16),
                         preferred_element_type=jnp.float32)
                 + bias_ref[b * 3 + l][None, :])
            outs.append(h)
        acc = acc + _selu(jnp.concatenate(outs, axis=1))

    macc = acc * mmul_ref[...]
    loc_ref[...] = (macc + madd_ref[...]).reshape(mb, 64, 384)
    glo_ref[...] = macc.reshape(mb, 64, 384).sum(axis=1)


@functools.partial(jax.jit, static_argnames=("interpret",))
def _run(x, adj_0, adj_1, adj_2, adj_3, madd, mmul, emb_pad, wcat, bias2,
         interpret=False):
    B = x.shape[0]
    grid = (B // MB,)
    blk = lambda *shape: pl.BlockSpec(shape, lambda i: (i,) + (0,) * (len(shape) - 1))
    full = lambda *shape: pl.BlockSpec(shape, lambda i: (0,) * len(shape))
    glo, loc = pl.pallas_call(
        _body,
        grid=grid,
        in_specs=[
            blk(MB, 64),
            blk(MB, 64, 64),
            blk(MB, 64, 64),
            blk(MB, 64, 64),
            blk(MB, 64, 64),
            blk(MB * 64, 1),
            blk(MB * 64, 1),
            full(128, 128),
            full(4, 3, 256, 128),
            full(12, 128),
        ],
        out_specs=[
            blk(MB, 384),
            blk(MB, 64, 384),
        ],
        out_shape=[
            jax.ShapeDtypeStruct((B, 384), jnp.float32),
            jax.ShapeDtypeStruct((B, 64, 384), jnp.float32),
        ],
        interpret=interpret,
    )(x, adj_0, adj_1, adj_2, adj_3, madd, mmul, emb_pad, wcat, bias2)
    return glo, loc


def kernel(x, adj_0, adj_1, adj_2, adj_3, mask, emb_table, W, Ws, bias):
    B, N = x.shape
    emb_pad = jnp.zeros((128, 128), jnp.float32).at[:emb_table.shape[0]].set(
        emb_table)
    wcat = jnp.concatenate([W, Ws], axis=2)
    bias2 = bias.reshape(12, 128)
    mcol = mask.reshape(B * N, 1)
    madd = jnp.where(mcol, 0.0, jnp.nan).astype(jnp.float32)
    mmul = mcol.astype(jnp.float32)
    glo, loc = _run(x.astype(jnp.int32), adj_0, adj_1, adj_2, adj_3,
                    madd, mmul, emb_pad, wcat, bias2)
    return glo, loc.reshape(B * N, 384)


# speedup vs baseline: 4.9533x; 1.0134x over previous
import functools

import jax
import jax.numpy as jnp
from jax import lax
from jax.experimental import pallas as pl

MB = 64

_SELU_SCALE = 1.0507009873554805
_SELU_ALPHA = 1.6732632423543772


def _selu(x):
    return _SELU_SCALE * jnp.where(x > 0, x, _SELU_ALPHA * (jnp.exp(x) - 1.0))


def _body(x_ref, a0_ref, a1_ref, a2_ref, a3_ref, madd_ref, mmul_ref, emb_ref,
          wcat_ref, bias_ref, glo_ref, loc_ref):
    mb = x_ref.shape[0]
    M = mb * 64
    xv = x_ref[...]
    iota = lax.broadcasted_iota(jnp.int32, (mb, 64, 128), 2)
    oh = (xv[:, :, None] == iota).astype(jnp.bfloat16)
    h0 = jnp.dot(oh.reshape(M, 128), emb_ref[...].astype(jnp.bfloat16),
                 preferred_element_type=jnp.float32)

    acc = jnp.zeros((M, 384), jnp.float32)
    for b, a_ref in enumerate((a0_ref, a1_ref, a2_ref, a3_ref)):
        A = a_ref[...].astype(jnp.bfloat16)
        h = h0
        outs = []
        for l in range(3):
            hb = h.astype(jnp.bfloat16)
            h3 = hb.reshape(mb, 64, 128)
            agg = lax.dot_general(
                A, h3,
                dimension_numbers=(((2,), (1,)), ((0,), (0,))),
                preferred_element_type=jnp.float32)
            hcat = jnp.concatenate(
                [agg.astype(jnp.bfloat16).reshape(M, 128), hb], axis=1)
            h = (jnp.dot(hcat, wcat_ref[b, l].astype(jnp.bfloat16),
                         preferred_element_type=jnp.float32)
                 + bias_ref[b * 3 + l][None, :])
            outs.append(h)
        acc = acc + _selu(jnp.concatenate(outs, axis=1))

    macc = acc * mmul_ref[...]
    loc_ref[...] = (macc + madd_ref[...]).reshape(mb, 64, 384)
    glo_ref[...] = macc.reshape(mb, 64, 384).sum(axis=1)


@functools.partial(jax.jit, static_argnames=("interpret",))
def _run(x, adj_0, adj_1, adj_2, adj_3, madd, mmul, emb_pad, wcat, bias2,
         interpret=False):
    B = x.shape[0]
    grid = (B // MB,)
    blk = lambda *shape: pl.BlockSpec(shape, lambda i: (i,) + (0,) * (len(shape) - 1))
    full = lambda *shape: pl.BlockSpec(shape, lambda i: (0,) * len(shape))
    glo, loc = pl.pallas_call(
        _body,
        grid=grid,
        in_specs=[
            blk(MB, 64),
            blk(MB, 64, 64),
            blk(MB, 64, 64),
            blk(MB, 64, 64),
            blk(MB, 64, 64),
            blk(MB * 64, 1),
            blk(MB * 64, 1),
            full(128, 128),
            full(4, 3, 256, 128),
            full(12, 128),
        ],
        out_specs=[
            blk(MB, 384),
            blk(MB, 64, 384),
        ],
        out_shape=[
            jax.ShapeDtypeStruct((B, 384), jnp.float32),
            jax.ShapeDtypeStruct((B, 64, 384), jnp.float32),
        ],
        interpret=interpret,
    )(x, adj_0, adj_1, adj_2, adj_3, madd, mmul, emb_pad, wcat, bias2)
    return glo, loc


def kernel(x, adj_0, adj_1, adj_2, adj_3, mask, emb_table, W, Ws, bias):
    B, N = x.shape
    emb_pad = jnp.zeros((128, 128), jnp.float32).at[:emb_table.shape[0]].set(
        emb_table)
    wcat = jnp.concatenate([W, Ws], axis=2)
    bias2 = bias.reshape(12, 128)
    mcol = mask.reshape(B * N, 1)
    madd = jnp.where(mcol, 0.0, jnp.nan).astype(jnp.float32)
    mmul = mcol.astype(jnp.float32)
    glo, loc = _run(x.astype(jnp.int32), adj_0, adj_1, adj_2, adj_3,
                    madd, mmul, emb_pad, wcat, bias2)
    return glo, loc.reshape(B * N, 384)
